# Initial kernel scaffold; baseline (speedup 1.0000x reference)
import jax
import jax.numpy as jnp
from jax.experimental import pallas as pl


def kernel(x, edge_index, decode_index, W1, b1, W2, b2):
    raise NotImplementedError("write your pallas kernel here")


# speedup vs baseline: 62.7462x; 62.7462x over previous
import functools

import jax
import jax.numpy as jnp
from jax import lax
from jax.experimental import pallas as pl
from jax.experimental.pallas import tpu as pltpu
from jax.experimental.pallas import tpu_sc as plsc

N = 10000
E = 320000
F_IN = 128
H = 16
Q = 100000

NC = 2
NS = 16
NW = NC * NS
EPW = E // NW
NPAD = 10240
ROWS_PER_SUB = NPAD // NS
ECHUNK = 125
NECHUNK = EPW // ECHUNK
NBUF = 8
NLAG = 4
QSETS = 4
QPAD = 102400
QPW = QPAD // NW
QCHUNK = 128
NQCHUNK = QPW // QCHUNK

_mesh = plsc.VectorSubcoreMesh(
    core_axis_name="c", subcore_axis_name="s", num_cores=NC, num_subcores=NS)
_sc_params = pltpu.CompilerParams(
    needs_layout_passes=False, use_tc_tiling_on_sc=False)


def _deg_body(dst_hbm, hv_hbm, histp_hbm, dst_v, hist_v):
  c = lax.axis_index("c")
  s = lax.axis_index("s")
  w = c * NS + s
  pltpu.sync_copy(dst_hbm.at[pl.ds(w * EPW, EPW)], dst_v)
  zeros16 = jnp.zeros((16,), jnp.float32)
  ones16 = jnp.ones((16,), jnp.float32)

  def zero_body(i, carry):
    hist_v[pl.ds(i * 16, 16)] = zeros16
    return carry
  lax.fori_loop(0, NPAD // 16, zero_body, 0, unroll=8)

  def acc_body(i, carry):
    d = dst_v[pl.ds(i * 16, 16)]
    plsc.addupdate_scatter(hist_v, [d], ones16)
    return carry
  lax.fori_loop(0, EPW // 16, acc_body, 0, unroll=8)
  pltpu.sync_copy(hist_v, histp_hbm.at[w])


_deg_call = pl.kernel(
    _deg_body,
    out_type=jax.ShapeDtypeStruct((NW, NPAD), jnp.float32),
    mesh=_mesh,
    compiler_params=_sc_params,
    scratch_types=[
        pltpu.VMEM((EPW,), jnp.int32),
        pltpu.VMEM((NPAD,), jnp.float32),
    ],
)


def _agg_body(hs_hbm, src2_hbm, dst2_hbm, zeros_hbm, aggp_hbm,
              hs_s, agg_s, sidx_v, didx_v, rows_v,
              g0, g1, g2, g3, g4, g5, g6, g7,
              s0, s1, s2, s3, s4, s5, s6, s7):
  gsems = (g0, g1, g2, g3, g4, g5, g6, g7)
  ssems = (s0, s1, s2, s3, s4, s5, s6, s7)
  c = lax.axis_index("c")
  s = lax.axis_index("s")
  w = c * NS + s
  rows = pl.ds(s * ROWS_PER_SUB, ROWS_PER_SUB)
  pltpu.sync_copy(hs_hbm.at[rows], hs_s.at[rows])
  pltpu.sync_copy(zeros_hbm.at[rows], agg_s.at[rows])
  erows = pl.ds(w * NECHUNK, NECHUNK)
  pltpu.sync_copy(src2_hbm.at[erows], sidx_v)
  pltpu.sync_copy(dst2_hbm.at[erows], didx_v)
  plsc.subcore_barrier()

  def gather_start(j, b):
    pltpu.async_copy(hs_s.at[sidx_v.at[j]], rows_v.at[b], gsems[b])

  def gather_wait(b):
    pltpu.make_async_copy(hs_s.at[sidx_v.at[0]], rows_v.at[b], gsems[b]).wait()

  def scatter_start(j, b):
    pltpu.async_copy(rows_v.at[b], agg_s.at[didx_v.at[j]], ssems[b], add=True)

  def scatter_wait(b):
    pltpu.make_async_copy(rows_v.at[b], agg_s.at[didx_v.at[0]],
                          ssems[b]).wait()

  for b in range(NLAG):
    gather_start(b, b)

  n_outer = NECHUNK // NBUF

  def body(i, carry):
    for b in range(NBUF):
      j = i * NBUF + b
      gather_wait(b)
      scatter_start(j, b)
      jj = j + NLAG
      bj = (b + NLAG) % NBUF
      @pl.when(jj < NECHUNK)
      def _():
        @pl.when(jj >= NBUF)
        def _():
          scatter_wait(bj)
        gather_start(jj, bj)
    return carry
  lax.fori_loop(0, n_outer, body, 0)
  for b in range(NBUF):
    scatter_wait(b)

  plsc.subcore_barrier()
  pltpu.sync_copy(agg_s.at[rows], aggp_hbm.at[c, rows])


_agg_call = pl.kernel(
    _agg_body,
    out_type=jax.ShapeDtypeStruct((NC, NPAD, H), jnp.float32),
    mesh=_mesh,
    compiler_params=_sc_params,
    scratch_types=[
        pltpu.VMEM_SHARED((NPAD, H), jnp.float32),
        pltpu.VMEM_SHARED((NPAD, H), jnp.float32),
        pltpu.VMEM((NECHUNK, ECHUNK), jnp.int32),
        pltpu.VMEM((NECHUNK, ECHUNK), jnp.int32),
        pltpu.VMEM((NBUF, ECHUNK, H), jnp.float32),
    ] + [pltpu.SemaphoreType.DMA] * (2 * NBUF),
)


def _newton_rsqrt(x):
  i = plsc.bitcast(x, jnp.int32)
  y = plsc.bitcast(jnp.int32(0x5F3759DF) - (i >> 1), jnp.float32)
  for _ in range(3):
    y = y * (1.5 - 0.5 * x * y * y)
  return y


def _decode_body(aggp_hbm, hs2_hbm, disb_hbm, b2_hbm, ia2_hbm, ib2_hbm,
                 out_hbm, g_s, w0_v, w1_v, w2_v, w3_v, sq_v, b2_v,
                 ia_v, ib_v, a_v, b_v, pp_v, o_v, q0, q1, q2, q3):
  qsems = (q0, q1, q2, q3)
  c = lax.axis_index("c")
  s = lax.axis_index("s")
  w = c * NS + s
  rows = pl.ds(s * ROWS_PER_SUB, ROWS_PER_SUB)
  qrows = pl.ds(w * NQCHUNK, NQCHUNK)
  stage = [
      pltpu.async_copy(aggp_hbm.at[0, rows], w0_v, q0),
      pltpu.async_copy(aggp_hbm.at[1, rows], w1_v, q0),
      pltpu.async_copy(hs2_hbm.at[rows], w2_v, q0),
      pltpu.async_copy(disb_hbm.at[rows], w3_v, q0),
      pltpu.async_copy(b2_hbm, b2_v, q0),
      pltpu.async_copy(ia2_hbm.at[qrows], ia_v, q0),
      pltpu.async_copy(ib2_hbm.at[qrows], ib_v, q0),
  ]
  for d in stage:
    d.wait()

  iota16 = lax.iota(jnp.int32, 16)
  iota17 = iota16 * 17
  b2row = b2_v[...]

  def rowfix(i, carry):
    h2f = w3_v[i] * (w0_v[i] + w1_v[i] + w2_v[i]) + b2row
    w0_v[i] = h2f
    sq_v[pl.ds(i * 17, 16)] = h2f * h2f
    return carry
  lax.fori_loop(0, ROWS_PER_SUB, rowfix, 0, unroll=4)

  def normbody(t, carry):
    accs = [jnp.zeros((16,), jnp.float32) for _ in range(4)]
    for f in range(H):
      idxf = t * 272 + f + iota17
      cf = plsc.load_gather(sq_v, [idxf])
      accs[f % 4] = accs[f % 4] + cf
    sa = (accs[0] + accs[1]) + (accs[2] + accs[3])
    rv16 = _newton_rsqrt(jnp.maximum(sa, 1e-12))
    for f in range(H):
      idxf = t * 272 + f + iota17
      plsc.store_scatter(sq_v, [idxf], rv16)
    return carry
  lax.fori_loop(0, ROWS_PER_SUB // 16, normbody, 0)

  def scalerow(i, carry):
    w0_v[i] = w0_v[i] * sq_v[pl.ds(i * 17, 16)]
    return carry
  lax.fori_loop(0, ROWS_PER_SUB, scalerow, 0, unroll=4)
  pltpu.sync_copy(w0_v, g_s.at[rows])
  plsc.subcore_barrier()

  def gather_start(j, p):
    pltpu.async_copy(g_s.at[ia_v.at[j]], a_v.at[p], qsems[p])
    pltpu.async_copy(g_s.at[ib_v.at[j]], b_v.at[p], qsems[p])

  def gather_wait(p):
    pltpu.make_async_copy(g_s.at[ia_v.at[0]], a_v.at[p], qsems[p]).wait()
    pltpu.make_async_copy(g_s.at[ib_v.at[0]], b_v.at[p], qsems[p]).wait()

  def compute(j, p):
    gather_wait(p)
    ap = a_v.at[p]
    bp = b_v.at[p]

    for i in range(QCHUNK):
      pp_v[pl.ds(i * 17, 16)] = ap[i] * bp[i]
    for t in range(QCHUNK // 16):
      accs = [jnp.zeros((16,), jnp.float32) for _ in range(4)]
      for f in range(H):
        idxf = t * 272 + f + iota17
        accs[f % 4] = accs[f % 4] + plsc.load_gather(pp_v, [idxf])
      acc = (accs[0] + accs[1]) + (accs[2] + accs[3])
      sig = 1.0 / (1.0 + jnp.exp(-acc))
      o_v[pl.ds(j * QCHUNK + t * 16, 16)] = sig

  for p in range(QSETS):
    gather_start(p, p)

  def body(i, carry):
    for p in range(QSETS):
      j = i * QSETS + p
      compute(j, p)
      @pl.when(j < NQCHUNK - QSETS)
      def _():
        gather_start(j + QSETS, p)
    return carry
  lax.fori_loop(0, NQCHUNK // QSETS, body, 0)
  compute(NQCHUNK - 1, 0)

  pltpu.sync_copy(o_v, out_hbm.at[pl.ds(w * QPW, QPW)])


_decode_call = pl.kernel(
    _decode_body,
    out_type=jax.ShapeDtypeStruct((QPAD,), jnp.float32),
    mesh=_mesh,
    compiler_params=_sc_params,
    scratch_types=[
        pltpu.VMEM_SHARED((NPAD, H), jnp.float32),
        pltpu.VMEM((ROWS_PER_SUB, H), jnp.float32),
        pltpu.VMEM((ROWS_PER_SUB, H), jnp.float32),
        pltpu.VMEM((ROWS_PER_SUB, H), jnp.float32),
        pltpu.VMEM((ROWS_PER_SUB, H), jnp.float32),
        pltpu.VMEM((ROWS_PER_SUB * 17,), jnp.float32),
        pltpu.VMEM((H,), jnp.float32),
        pltpu.VMEM((NQCHUNK, QCHUNK), jnp.int32),
        pltpu.VMEM((NQCHUNK, QCHUNK), jnp.int32),
        pltpu.VMEM((QSETS, QCHUNK, H), jnp.float32),
        pltpu.VMEM((QSETS, QCHUNK, H), jnp.float32),
        pltpu.VMEM((QCHUNK * 17,), jnp.float32),
        pltpu.VMEM((QPW,), jnp.float32),
    ] + [pltpu.SemaphoreType.DMA] * QSETS,
)


VN = N // 8
VNP = NPAD // 8


def _mm_body(x8_ref, w1e_ref, hv_ref):
  hv_ref[...] = lax.dot(x8_ref[...], w1e_ref[...],
                        precision=lax.Precision.HIGHEST)


_mm_call = pl.pallas_call(
    _mm_body,
    out_shape=jax.ShapeDtypeStruct((VN, 128), jnp.float32),
)


def _scale_body(hv_ref, histp_ref, hs1_ref, disb_ref):
  ones_col = jnp.ones((NW, 1), jnp.float32)
  deg = lax.dot_general(histp_ref[...], ones_col, (((0,), (0,)), ((), ())),
                        precision=lax.Precision.DEFAULT) + 1.0
  dis = lax.rsqrt(deg)
  disb = jnp.reshape(
      jnp.broadcast_to(jnp.reshape(dis, (VNP, 8, 1)), (VNP, 8, H)),
      (VNP, 128))
  disb_ref[...] = disb
  hs1_ref[...] = jnp.concatenate(
      [hv_ref[...] * disb[:VN],
       jnp.zeros((VNP - VN, 128), jnp.float32)], axis=0)


_scale_call = pl.pallas_call(
    _scale_body,
    out_shape=[
        jax.ShapeDtypeStruct((VNP, 128), jnp.float32),
        jax.ShapeDtypeStruct((VNP, 128), jnp.float32),
    ],
)


def _dense2_body(aggp_ref, hs1_ref, disb_ref, b1t_ref, w2bd_ref, hs2_ref):
  agg = aggp_ref[0] + aggp_ref[1]
  disb = disb_ref[...]
  out1 = jnp.maximum(disb * (agg + hs1_ref[...]) + b1t_ref[...][None, :], 0.0)
  h2 = lax.dot(out1, w2bd_ref[...], precision=lax.Precision.HIGHEST)
  hs2_ref[...] = h2 * disb


_dense2_call = pl.pallas_call(
    _dense2_body,
    out_shape=jax.ShapeDtypeStruct((VNP, 128), jnp.float32),
)


@jax.jit
def kernel(x, edge_index, decode_index, W1, b1, W2, b2):
  src = edge_index[0]
  dst = edge_index[1]
  src2 = src.reshape(E // ECHUNK, ECHUNK)
  dst2 = dst.reshape(E // ECHUNK, ECHUNK)
  eye8 = jnp.eye(8, dtype=jnp.float32)
  w1e = jnp.kron(eye8, W1)
  w2bd = jnp.kron(eye8, W2)
  b1t = jnp.tile(b1, 8)
  hv = _mm_call(x.reshape(VN, 8 * F_IN), w1e)
  histp = _deg_call(dst, hv)
  hs1v, disbv = _scale_call(hv, histp)
  hs1 = hs1v.reshape(NPAD, H)
  zeros_nh = jnp.zeros((NPAD, H), jnp.float32)
  aggp1 = _agg_call(hs1, src2, dst2, zeros_nh)
  hs2v = _dense2_call(aggp1.reshape(NC, VNP, 128), hs1v, disbv, b1t, w2bd)
  hs2 = hs2v.reshape(NPAD, H)
  aggp2 = _agg_call(hs2, src2, dst2, zeros_nh)
  pad = jnp.zeros((QPAD - Q,), jnp.int32)
  ia2 = jnp.concatenate([decode_index[0], pad]).reshape(QPAD // QCHUNK, QCHUNK)
  ib2 = jnp.concatenate([decode_index[1], pad]).reshape(QPAD // QCHUNK, QCHUNK)
  out = _decode_call(aggp2, hs2, disbv.reshape(NPAD, H), b2, ia2, ib2)
  return out[:Q]


# speedup vs baseline: 65.8467x; 1.0494x over previous
import functools

import jax
import jax.numpy as jnp
from jax import lax
from jax.experimental import pallas as pl
from jax.experimental.pallas import tpu as pltpu
from jax.experimental.pallas import tpu_sc as plsc

N = 10000
E = 320000
F_IN = 128
H = 16
Q = 100000

NC = 2
NS = 16
NW = NC * NS
EPW = E // NW
NPAD = 10240
ROWS_PER_SUB = NPAD // NS
ECHUNK = 125
NECHUNK = EPW // ECHUNK
NBUF = 8
NLAG = 4
QSETS = 4
QPAD = 102400
QPW = QPAD // NW
QCHUNK = 128
NQCHUNK = QPW // QCHUNK

_mesh = plsc.VectorSubcoreMesh(
    core_axis_name="c", subcore_axis_name="s", num_cores=NC, num_subcores=NS)
_sc_params = pltpu.CompilerParams(
    needs_layout_passes=False, use_tc_tiling_on_sc=False)


def _deg_body(dst_hbm, hv_hbm, histp_hbm, dst_v, hist_v):
  c = lax.axis_index("c")
  s = lax.axis_index("s")
  w = c * NS + s
  pltpu.sync_copy(dst_hbm.at[pl.ds(w * EPW, EPW)], dst_v)
  zeros16 = jnp.zeros((16,), jnp.float32)
  ones16 = jnp.ones((16,), jnp.float32)

  def zero_body(i, carry):
    hist_v[pl.ds(i * 16, 16)] = zeros16
    return carry
  lax.fori_loop(0, NPAD // 16, zero_body, 0, unroll=8)

  def acc_body(i, carry):
    d = dst_v[pl.ds(i * 16, 16)]
    plsc.addupdate_scatter(hist_v, [d], ones16)
    return carry
  lax.fori_loop(0, EPW // 16, acc_body, 0, unroll=8)
  pltpu.sync_copy(hist_v, histp_hbm.at[w])


_deg_call = pl.kernel(
    _deg_body,
    out_type=jax.ShapeDtypeStruct((NW, NPAD), jnp.float32),
    mesh=_mesh,
    compiler_params=_sc_params,
    scratch_types=[
        pltpu.VMEM((EPW,), jnp.int32),
        pltpu.VMEM((NPAD,), jnp.float32),
    ],
)


def _agg_body(hs_hbm, src2_hbm, dst2_hbm, aggp_hbm,
              hs_s, agg_s, sidx_v, didx_v, rows_v, zb_v,
              g0, g1, g2, g3, g4, g5, g6, g7,
              s0, s1, s2, s3, s4, s5, s6, s7):
  gsems = (g0, g1, g2, g3, g4, g5, g6, g7)
  ssems = (s0, s1, s2, s3, s4, s5, s6, s7)
  c = lax.axis_index("c")
  s = lax.axis_index("s")
  w = c * NS + s
  rows = pl.ds(s * ROWS_PER_SUB, ROWS_PER_SUB)
  erows = pl.ds(w * NECHUNK, NECHUNK)
  stage = [
      pltpu.async_copy(hs_hbm.at[rows], hs_s.at[rows], g0),
      pltpu.async_copy(src2_hbm.at[erows], sidx_v, g0),
      pltpu.async_copy(dst2_hbm.at[erows], didx_v, g0),
  ]
  zeros16 = jnp.zeros((16,), jnp.float32)

  def zrow(i, carry):
    zb_v[i] = zeros16
    return carry
  lax.fori_loop(0, ROWS_PER_SUB, zrow, 0, unroll=8)
  pltpu.sync_copy(zb_v, agg_s.at[rows])
  for d in stage:
    d.wait()
  plsc.subcore_barrier()

  def gather_start(j, b):
    pltpu.async_copy(hs_s.at[sidx_v.at[j]], rows_v.at[b], gsems[b])

  def gather_wait(b):
    pltpu.make_async_copy(hs_s.at[sidx_v.at[0]], rows_v.at[b], gsems[b]).wait()

  def scatter_start(j, b):
    pltpu.async_copy(rows_v.at[b], agg_s.at[didx_v.at[j]], ssems[b], add=True)

  def scatter_wait(b):
    pltpu.make_async_copy(rows_v.at[b], agg_s.at[didx_v.at[0]],
                          ssems[b]).wait()

  for b in range(NLAG):
    gather_start(b, b)

  n_outer = NECHUNK // NBUF

  def body(i, carry):
    for b in range(NBUF):
      j = i * NBUF + b
      gather_wait(b)
      scatter_start(j, b)
      jj = j + NLAG
      bj = (b + NLAG) % NBUF
      @pl.when(jj < NECHUNK)
      def _():
        @pl.when(jj >= NBUF)
        def _():
          scatter_wait(bj)
        gather_start(jj, bj)
    return carry
  lax.fori_loop(0, n_outer, body, 0)
  for b in range(NBUF):
    scatter_wait(b)

  plsc.subcore_barrier()
  pltpu.sync_copy(agg_s.at[rows], aggp_hbm.at[c, rows])


_agg_call = pl.kernel(
    _agg_body,
    out_type=jax.ShapeDtypeStruct((NC, NPAD, H), jnp.float32),
    mesh=_mesh,
    compiler_params=_sc_params,
    scratch_types=[
        pltpu.VMEM_SHARED((NPAD, H), jnp.float32),
        pltpu.VMEM_SHARED((NPAD, H), jnp.float32),
        pltpu.VMEM((NECHUNK, ECHUNK), jnp.int32),
        pltpu.VMEM((NECHUNK, ECHUNK), jnp.int32),
        pltpu.VMEM((NBUF, ECHUNK, H), jnp.float32),
        pltpu.VMEM((ROWS_PER_SUB, H), jnp.float32),
    ] + [pltpu.SemaphoreType.DMA] * (2 * NBUF),
)


def _newton_rsqrt(x):
  i = plsc.bitcast(x, jnp.int32)
  y = plsc.bitcast(jnp.int32(0x5F3759DF) - (i >> 1), jnp.float32)
  for _ in range(3):
    y = y * (1.5 - 0.5 * x * y * y)
  return y


def _decode_body(aggp_hbm, hs2_hbm, disb_hbm, b2_hbm, ia2_hbm, ib2_hbm,
                 out_hbm, g_s, w0_v, w1_v, w2_v, w3_v, sq_v, b2_v,
                 ia_v, ib_v, a_v, b_v, pp_v, o_v, q0, q1, q2, q3):
  qsems = (q0, q1, q2, q3)
  c = lax.axis_index("c")
  s = lax.axis_index("s")
  w = c * NS + s
  rows = pl.ds(s * ROWS_PER_SUB, ROWS_PER_SUB)
  qrows = pl.ds(w * NQCHUNK, NQCHUNK)
  stage = [
      pltpu.async_copy(aggp_hbm.at[0, rows], w0_v, q0),
      pltpu.async_copy(aggp_hbm.at[1, rows], w1_v, q0),
      pltpu.async_copy(hs2_hbm.at[rows], w2_v, q0),
      pltpu.async_copy(disb_hbm.at[rows], w3_v, q0),
      pltpu.async_copy(b2_hbm, b2_v, q0),
      pltpu.async_copy(ia2_hbm.at[qrows], ia_v, q0),
      pltpu.async_copy(ib2_hbm.at[qrows], ib_v, q0),
  ]
  for d in stage:
    d.wait()

  iota16 = lax.iota(jnp.int32, 16)
  iota17 = iota16 * 17
  b2row = b2_v[...]

  def rowfix(i, carry):
    h2f = w3_v[i] * (w0_v[i] + w1_v[i] + w2_v[i]) + b2row
    w0_v[i] = h2f
    sq_v[pl.ds(i * 17, 16)] = h2f * h2f
    return carry
  lax.fori_loop(0, ROWS_PER_SUB, rowfix, 0, unroll=4)

  def normbody(t, carry):
    accs = [jnp.zeros((16,), jnp.float32) for _ in range(4)]
    for f in range(H):
      idxf = t * 272 + f + iota17
      cf = plsc.load_gather(sq_v, [idxf])
      accs[f % 4] = accs[f % 4] + cf
    sa = (accs[0] + accs[1]) + (accs[2] + accs[3])
    rv16 = _newton_rsqrt(jnp.maximum(sa, 1e-12))
    for f in range(H):
      idxf = t * 272 + f + iota17
      plsc.store_scatter(sq_v, [idxf], rv16)
    return carry
  lax.fori_loop(0, ROWS_PER_SUB // 16, normbody, 0)

  def scalerow(i, carry):
    w0_v[i] = w0_v[i] * sq_v[pl.ds(i * 17, 16)]
    return carry
  lax.fori_loop(0, ROWS_PER_SUB, scalerow, 0, unroll=4)
  pltpu.sync_copy(w0_v, g_s.at[rows])
  plsc.subcore_barrier()

  def gather_start(j, p):
    pltpu.async_copy(g_s.at[ia_v.at[j]], a_v.at[p], qsems[p])
    pltpu.async_copy(g_s.at[ib_v.at[j]], b_v.at[p], qsems[p])

  def gather_wait(p):
    pltpu.make_async_copy(g_s.at[ia_v.at[0]], a_v.at[p], qsems[p]).wait()
    pltpu.make_async_copy(g_s.at[ib_v.at[0]], b_v.at[p], qsems[p]).wait()

  def compute(j, p):
    gather_wait(p)
    ap = a_v.at[p]
    bp = b_v.at[p]

    for i in range(QCHUNK):
      pp_v[pl.ds(i * 17, 16)] = ap[i] * bp[i]
    for t in range(QCHUNK // 16):
      accs = [jnp.zeros((16,), jnp.float32) for _ in range(4)]
      for f in range(H):
        idxf = t * 272 + f + iota17
        accs[f % 4] = accs[f % 4] + plsc.load_gather(pp_v, [idxf])
      acc = (accs[0] + accs[1]) + (accs[2] + accs[3])
      sig = 1.0 / (1.0 + jnp.exp(-acc))
      o_v[pl.ds(j * QCHUNK + t * 16, 16)] = sig

  for p in range(QSETS):
    gather_start(p, p)

  def body(i, carry):
    for p in range(QSETS):
      j = i * QSETS + p
      compute(j, p)
      @pl.when(j < NQCHUNK - QSETS)
      def _():
        gather_start(j + QSETS, p)
    return carry
  lax.fori_loop(0, NQCHUNK // QSETS, body, 0)
  compute(NQCHUNK - 1, 0)

  pltpu.sync_copy(o_v, out_hbm.at[pl.ds(w * QPW, QPW)])


_decode_call = pl.kernel(
    _decode_body,
    out_type=jax.ShapeDtypeStruct((QPAD,), jnp.float32),
    mesh=_mesh,
    compiler_params=_sc_params,
    scratch_types=[
        pltpu.VMEM_SHARED((NPAD, H), jnp.float32),
        pltpu.VMEM((ROWS_PER_SUB, H), jnp.float32),
        pltpu.VMEM((ROWS_PER_SUB, H), jnp.float32),
        pltpu.VMEM((ROWS_PER_SUB, H), jnp.float32),
        pltpu.VMEM((ROWS_PER_SUB, H), jnp.float32),
        pltpu.VMEM((ROWS_PER_SUB * 17,), jnp.float32),
        pltpu.VMEM((H,), jnp.float32),
        pltpu.VMEM((NQCHUNK, QCHUNK), jnp.int32),
        pltpu.VMEM((NQCHUNK, QCHUNK), jnp.int32),
        pltpu.VMEM((QSETS, QCHUNK, H), jnp.float32),
        pltpu.VMEM((QSETS, QCHUNK, H), jnp.float32),
        pltpu.VMEM((QCHUNK * 17,), jnp.float32),
        pltpu.VMEM((QPW,), jnp.float32),
    ] + [pltpu.SemaphoreType.DMA] * QSETS,
)


VN = N // 8
VNP = NPAD // 8


def _blockdiag(w_ref, reps, rows, cols):
  t = jnp.tile(w_ref[...], (reps, reps))
  ri = lax.broadcasted_iota(jnp.int32, (reps * rows, reps * cols), 0)
  ci = lax.broadcasted_iota(jnp.int32, (reps * rows, reps * cols), 1)
  return jnp.where(ri // rows == ci // cols, t, 0.0)


def _mm_body(x8_ref, w1_ref, hv_ref):
  hv_ref[...] = lax.dot(x8_ref[...], _blockdiag(w1_ref, 8, F_IN, H),
                        precision=lax.Precision.HIGHEST)


_mm_call = pl.pallas_call(
    _mm_body,
    out_shape=jax.ShapeDtypeStruct((VN, 128), jnp.float32),
)


def _scale_body(hv_ref, histp_ref, hs1_ref, disb_ref):
  ones_col = jnp.ones((NW, 1), jnp.float32)
  deg = lax.dot_general(histp_ref[...], ones_col, (((0,), (0,)), ((), ())),
                        precision=lax.Precision.DEFAULT) + 1.0
  dis = lax.rsqrt(deg)
  disb = jnp.reshape(
      jnp.broadcast_to(jnp.reshape(dis, (VNP, 8, 1)), (VNP, 8, H)),
      (VNP, 128))
  disb_ref[...] = disb
  hs1_ref[...] = jnp.concatenate(
      [hv_ref[...] * disb[:VN],
       jnp.zeros((VNP - VN, 128), jnp.float32)], axis=0)


_scale_call = pl.pallas_call(
    _scale_body,
    out_shape=[
        jax.ShapeDtypeStruct((VNP, 128), jnp.float32),
        jax.ShapeDtypeStruct((VNP, 128), jnp.float32),
    ],
)


def _dense2_body(aggp_ref, hs1_ref, disb_ref, b1_ref, w2_ref, hs2_ref):
  agg = aggp_ref[0] + aggp_ref[1]
  disb = disb_ref[...]
  b1t = jnp.tile(b1_ref[...], 8)
  out1 = jnp.maximum(disb * (agg + hs1_ref[...]) + b1t[None, :], 0.0)
  h2 = lax.dot(out1, _blockdiag(w2_ref, 8, H, H),
               precision=lax.Precision.HIGHEST)
  hs2_ref[...] = h2 * disb


_dense2_call = pl.pallas_call(
    _dense2_body,
    out_shape=jax.ShapeDtypeStruct((VNP, 128), jnp.float32),
)


@jax.jit
def kernel(x, edge_index, decode_index, W1, b1, W2, b2):
  src = edge_index[0]
  dst = edge_index[1]
  src2 = src.reshape(E // ECHUNK, ECHUNK)
  dst2 = dst.reshape(E // ECHUNK, ECHUNK)
  hv = _mm_call(x.reshape(VN, 8 * F_IN), W1)
  histp = _deg_call(dst, hv)
  hs1v, disbv = _scale_call(hv, histp)
  hs1 = hs1v.reshape(NPAD, H)
  aggp1 = _agg_call(hs1, src2, dst2)
  hs2v = _dense2_call(aggp1.reshape(NC, VNP, 128), hs1v, disbv, b1, W2)
  hs2 = hs2v.reshape(NPAD, H)
  aggp2 = _agg_call(hs2, src2, dst2)
  pad = jnp.zeros((QPAD - Q,), jnp.int32)
  ia2 = jnp.concatenate([decode_index[0], pad]).reshape(QPAD // QCHUNK, QCHUNK)
  ib2 = jnp.concatenate([decode_index[1], pad]).reshape(QPAD // QCHUNK, QCHUNK)
  out = _decode_call(aggp2, hs2, disbv.reshape(NPAD, H), b2, ia2, ib2)
  return out[:Q]
